# baseline (speedup 1.0000x reference)
import jax
import jax.numpy as jnp
from jax.experimental import pallas as pl
from jax.experimental.pallas import tpu as pltpu

D = 2048
SK = 8
NSCORE = SK * SK
TOP_K = 8
BLK = 1024

_SIGN_FIX = 0x7FFFFFFF
_LOW_MASK = ~(NSCORE - 1)
_KEY_MIN = -(2 ** 31)


def _router_kernel(x_ref, w_ref, small_ref, scores_ref):
    x = x_ref[...]
    w = w_ref[...]
    s = jax.lax.dot_general(
        x, w, (((1,), (1,)), ((), ())),
        preferred_element_type=jnp.float32)
    st = s.T
    s1t = st[:SK]
    s2t = st[SK:]
    scores = (s1t[:, None, :] + s2t[None, :, :]).reshape(NSCORE, BLK)
    scores_ref[...] = scores.T

    bits = jax.lax.bitcast_convert_type(scores, jnp.int32)
    sortable = jnp.where(bits >= 0, bits, bits ^ _SIGN_FIX)
    row = jax.lax.broadcasted_iota(jnp.int32, (NSCORE, BLK), 0)
    key = (sortable & _LOW_MASK) | (NSCORE - 1 - row)

    row8 = jax.lax.broadcasted_iota(jnp.int32, (TOP_K, BLK), 0)
    vals = jnp.zeros((TOP_K, BLK), jnp.float32)
    idxf = jnp.zeros((TOP_K, BLK), jnp.float32)
    for k in range(TOP_K):
        kmax = jnp.max(key, axis=0, keepdims=True)
        ki = ((NSCORE - 1) - (kmax & (NSCORE - 1))).astype(jnp.float32)
        vb = kmax & _LOW_MASK
        fb = jnp.where(vb >= 0, vb, vb ^ _SIGN_FIX)
        kv = jax.lax.bitcast_convert_type(fb, jnp.float32)
        idxf = jnp.where(row8 == k, ki, idxf)
        vals = jnp.where(row8 == k, kv, vals)
        key = jnp.where(key == kmax, _KEY_MIN, key)

    idx_as_f32 = jax.lax.bitcast_convert_type(
        idxf.astype(jnp.int32), jnp.float32)
    e = jnp.exp(vals - vals[:1])
    probs = e / jnp.sum(e, axis=0, keepdims=True)
    small_ref[...] = jnp.concatenate([idx_as_f32, probs], axis=0).T


@jax.jit
def kernel(x, W1, W2):
    n_tok = x.shape[0]
    w = jnp.concatenate([W1, W2], axis=0)
    grid = (n_tok // BLK,)
    out = pl.pallas_call(
        _router_kernel,
        grid=grid,
        in_specs=[
            pl.BlockSpec((BLK, D), lambda i: (i, 0)),
            pl.BlockSpec((2 * SK, D), lambda i: (0, 0)),
        ],
        out_specs=[
            pl.BlockSpec((BLK, 2 * TOP_K), lambda i: (i, 0)),
            pl.BlockSpec((BLK, NSCORE), lambda i: (i, 0)),
        ],
        out_shape=[
            jax.ShapeDtypeStruct((n_tok, 2 * TOP_K), jnp.float32),
            jax.ShapeDtypeStruct((n_tok, NSCORE), jnp.float32),
        ],
    )(x, w)
    small, scores = out
    idx = jax.lax.bitcast_convert_type(small[:, :TOP_K], jnp.int32)
    probs = small[:, TOP_K:]
    return (idx, probs, scores)


# speedup vs baseline: 1.1006x; 1.1006x over previous
import jax
import jax.numpy as jnp
from jax.experimental import pallas as pl

D = 2048
SK = 8
NSCORE = SK * SK
TOP_K = 8
BLK = 1024


def _router_kernel(x_ref, w_ref, idx_ref, probs_ref, scores_ref):
    x = x_ref[...]
    w = w_ref[...]
    s = jax.lax.dot_general(
        x, w, (((1,), (1,)), ((), ())),
        preferred_element_type=jnp.float32)
    st = s.T
    s1t = st[:SK]
    s2t = st[SK:]
    scores = (s1t[:, None, :] + s2t[None, :, :]).reshape(NSCORE, BLK)
    scores_ref[...] = scores.T

    row = jax.lax.broadcasted_iota(jnp.int32, (NSCORE, BLK), 0)
    rev = (NSCORE - 1 - row).astype(jnp.float32)
    row8 = jax.lax.broadcasted_iota(jnp.int32, (TOP_K, BLK), 0)
    vals = jnp.zeros((TOP_K, BLK), jnp.float32)
    revs = jnp.zeros((TOP_K, BLK), jnp.float32)
    cur = scores
    neg_one = jnp.float32(-1.0)
    neg_inf = jnp.float32(-jnp.inf)
    for k in range(TOP_K):
        m = jnp.max(cur, axis=0, keepdims=True)
        sel = cur == m
        mi = jnp.max(jnp.where(sel, rev, neg_one), axis=0,
                     keepdims=True)
        vals = jnp.where(row8 == k, m, vals)
        revs = jnp.where(row8 == k, mi, revs)
        cur = jnp.where(sel & (rev == mi), neg_inf, cur)

    idx_ref[...] = (NSCORE - 1) - revs.T.astype(jnp.int32)
    e = jnp.exp(vals - vals[:1])
    probs_ref[...] = (e / jnp.sum(e, axis=0, keepdims=True)).T


@jax.jit
def kernel(x, W1, W2):
    n_tok = x.shape[0]
    w = jnp.concatenate([W1, W2], axis=0)
    grid = (n_tok // BLK,)
    out = pl.pallas_call(
        _router_kernel,
        grid=grid,
        in_specs=[
            pl.BlockSpec((BLK, D), lambda i: (i, 0)),
            pl.BlockSpec((2 * SK, D), lambda i: (0, 0)),
        ],
        out_specs=[
            pl.BlockSpec((BLK, TOP_K), lambda i: (i, 0)),
            pl.BlockSpec((BLK, TOP_K), lambda i: (i, 0)),
            pl.BlockSpec((BLK, NSCORE), lambda i: (i, 0)),
        ],
        out_shape=[
            jax.ShapeDtypeStruct((n_tok, TOP_K), jnp.int32),
            jax.ShapeDtypeStruct((n_tok, TOP_K), jnp.float32),
            jax.ShapeDtypeStruct((n_tok, NSCORE), jnp.float32),
        ],
    )(x, w)
    return (out[0], out[1], out[2])


# speedup vs baseline: 1.1052x; 1.0041x over previous
import jax
import jax.numpy as jnp
from jax.experimental import pallas as pl

D = 2048
SK = 8
NSCORE = SK * SK
TOP_K = 8
BLK = 1024


def _router_kernel(x_ref, w_ref, idx_ref, probs_ref, scores_ref):
    x = x_ref[...]
    w = w_ref[...]
    s = jax.lax.dot_general(
        x, w, (((1,), (1,)), ((), ())),
        preferred_element_type=jnp.float32)
    st = s.T
    s1t = st[:SK]
    s2t = st[SK:]
    scores = (s1t[:, None, :] + s2t[None, :, :]).reshape(NSCORE, BLK)
    scores_ref[...] = scores.T

    row = jax.lax.broadcasted_iota(jnp.int32, (NSCORE, BLK), 0)
    rev = (NSCORE - 1 - row).astype(jnp.float32)
    row8 = jax.lax.broadcasted_iota(jnp.int32, (TOP_K, BLK), 0)
    vals = jnp.zeros((TOP_K, BLK), jnp.float32)
    revs = jnp.zeros((TOP_K, BLK), jnp.float32)
    cur = scores
    neg_one = jnp.float32(-1.0)
    neg_inf = jnp.float32(-jnp.inf)
    for k in range(TOP_K):
        m = jnp.max(cur, axis=0, keepdims=True)
        sel = cur == m
        mi = jnp.max(jnp.where(sel, rev, neg_one), axis=0,
                     keepdims=True)
        vals = jnp.where(row8 == k, m, vals)
        revs = jnp.where(row8 == k, mi, revs)
        cur = jnp.where(sel & (rev == mi), neg_inf, cur)

    base = pl.program_id(0) * BLK
    idx_ref[pl.ds(base, BLK), :] = (NSCORE - 1) - revs.T.astype(jnp.int32)
    e = jnp.exp(vals - vals[:1])
    probs_ref[pl.ds(base, BLK), :] = (
        e / jnp.sum(e, axis=0, keepdims=True)).T


@jax.jit
def kernel(x, W1, W2):
    n_tok = x.shape[0]
    w = jnp.concatenate([W1, W2], axis=0)
    grid = (n_tok // BLK,)
    out = pl.pallas_call(
        _router_kernel,
        grid=grid,
        in_specs=[
            pl.BlockSpec((BLK, D), lambda i: (i, 0)),
            pl.BlockSpec((2 * SK, D), lambda i: (0, 0)),
        ],
        out_specs=[
            pl.BlockSpec((n_tok, TOP_K), lambda i: (0, 0)),
            pl.BlockSpec((n_tok, TOP_K), lambda i: (0, 0)),
            pl.BlockSpec((BLK, NSCORE), lambda i: (i, 0)),
        ],
        out_shape=[
            jax.ShapeDtypeStruct((n_tok, TOP_K), jnp.int32),
            jax.ShapeDtypeStruct((n_tok, TOP_K), jnp.float32),
            jax.ShapeDtypeStruct((n_tok, NSCORE), jnp.float32),
        ],
    )(x, w)
    return (out[0], out[1], out[2])


# speedup vs baseline: 1.3813x; 1.2498x over previous
import jax
import jax.numpy as jnp
from jax.experimental import pallas as pl

D = 2048
SK = 8
NSCORE = SK * SK
TOP_K = 8
BLK = 1024


def _router_kernel(x_ref, w_ref, idx_ref, probs_ref, scores_ref):
    x = x_ref[...]
    w = w_ref[...]
    s = jax.lax.dot_general(
        x, w, (((1,), (1,)), ((), ())),
        preferred_element_type=jnp.float32)
    st = s.T
    s1t = st[:SK]
    s2t = st[SK:]
    scores = (s1t[:, None, :] + s2t[None, :, :]).reshape(NSCORE, BLK)
    scores_ref[...] = scores.T

    row = jax.lax.broadcasted_iota(jnp.int32, (NSCORE, BLK), 0)
    rev = (NSCORE - 1 - row).astype(jnp.float32)
    row8 = jax.lax.broadcasted_iota(jnp.int32, (TOP_K, BLK), 0)
    vals = jnp.zeros((TOP_K, BLK), jnp.float32)
    revs = jnp.zeros((TOP_K, BLK), jnp.float32)
    cur = scores
    neg_one = jnp.float32(-1.0)
    neg_inf = jnp.float32(-jnp.inf)
    for k in range(TOP_K):
        m = jnp.max(cur, axis=0, keepdims=True)
        sel = cur == m
        mi = jnp.max(jnp.where(sel, rev, neg_one), axis=0,
                     keepdims=True)
        vals = jnp.where(row8 == k, m, vals)
        revs = jnp.where(row8 == k, mi, revs)
        cur = jnp.where(sel & (rev == mi), neg_inf, cur)

    idx_ref[...] = (NSCORE - 1) - revs.astype(jnp.int32)
    e = jnp.exp(vals - vals[:1])
    probs_ref[...] = e / jnp.sum(e, axis=0, keepdims=True)


@jax.jit
def kernel(x, W1, W2):
    n_tok = x.shape[0]
    w = jnp.concatenate([W1, W2], axis=0)
    grid = (n_tok // BLK,)
    out = pl.pallas_call(
        _router_kernel,
        grid=grid,
        in_specs=[
            pl.BlockSpec((BLK, D), lambda i: (i, 0)),
            pl.BlockSpec((2 * SK, D), lambda i: (0, 0)),
        ],
        out_specs=[
            pl.BlockSpec((TOP_K, BLK), lambda i: (0, i)),
            pl.BlockSpec((TOP_K, BLK), lambda i: (0, i)),
            pl.BlockSpec((BLK, NSCORE), lambda i: (i, 0)),
        ],
        out_shape=[
            jax.ShapeDtypeStruct((TOP_K, n_tok), jnp.int32),
            jax.ShapeDtypeStruct((TOP_K, n_tok), jnp.float32),
            jax.ShapeDtypeStruct((n_tok, NSCORE), jnp.float32),
        ],
    )(x, w)
    return (out[0].T, out[1].T, out[2])


# speedup vs baseline: 1.6017x; 1.1596x over previous
import jax
import jax.numpy as jnp
from jax.experimental import pallas as pl

D = 2048
SK = 8
NSCORE = SK * SK
TOP_K = 8
BLK = 1024


def _router_kernel(x_ref, w_ref, idx_ref, probs_ref, scores_ref):
    x = x_ref[...]
    w = w_ref[...]
    s = jax.lax.dot_general(
        x, w, (((1,), (1,)), ((), ())),
        preferred_element_type=jnp.float32)
    st = s.T
    s1t = st[:SK]
    s2t = st[SK:]
    scores = (s1t[:, None, :] + s2t[None, :, :]).reshape(NSCORE, BLK)
    scores_ref[...] = scores

    row = jax.lax.broadcasted_iota(jnp.int32, (NSCORE, BLK), 0)
    rev = (NSCORE - 1 - row).astype(jnp.float32)
    row8 = jax.lax.broadcasted_iota(jnp.int32, (TOP_K, BLK), 0)
    vals = jnp.zeros((TOP_K, BLK), jnp.float32)
    revs = jnp.zeros((TOP_K, BLK), jnp.float32)
    cur = scores
    neg_one = jnp.float32(-1.0)
    neg_inf = jnp.float32(-jnp.inf)
    for k in range(TOP_K):
        m = jnp.max(cur, axis=0, keepdims=True)
        sel = cur == m
        mi = jnp.max(jnp.where(sel, rev, neg_one), axis=0,
                     keepdims=True)
        vals = jnp.where(row8 == k, m, vals)
        revs = jnp.where(row8 == k, mi, revs)
        cur = jnp.where(sel & (rev == mi), neg_inf, cur)

    idx_ref[...] = (NSCORE - 1) - revs.astype(jnp.int32)
    e = jnp.exp(vals - vals[:1])
    probs_ref[...] = e / jnp.sum(e, axis=0, keepdims=True)


@jax.jit
def kernel(x, W1, W2):
    n_tok = x.shape[0]
    w = jnp.concatenate([W1, W2], axis=0)
    grid = (n_tok // BLK,)
    out = pl.pallas_call(
        _router_kernel,
        grid=grid,
        in_specs=[
            pl.BlockSpec((BLK, D), lambda i: (i, 0)),
            pl.BlockSpec((2 * SK, D), lambda i: (0, 0)),
        ],
        out_specs=[
            pl.BlockSpec((TOP_K, BLK), lambda i: (0, i)),
            pl.BlockSpec((TOP_K, BLK), lambda i: (0, i)),
            pl.BlockSpec((NSCORE, BLK), lambda i: (0, i)),
        ],
        out_shape=[
            jax.ShapeDtypeStruct((TOP_K, n_tok), jnp.int32),
            jax.ShapeDtypeStruct((TOP_K, n_tok), jnp.float32),
            jax.ShapeDtypeStruct((NSCORE, n_tok), jnp.float32),
        ],
    )(x, w)
    return (out[0].T, out[1].T, out[2].T)
